# baseline (speedup 1.0000x reference)
import functools

import jax
import jax.numpy as jnp
from jax import lax
from jax.experimental import pallas as pl
from jax.experimental.pallas import tpu as pltpu
from jax.experimental.pallas import tpu_sc as plsc

NB = 8
DEPTH = 3
TAB_REP = 4


def _sc_embed_add(xr, posf, tab):
    nr, e = xr.shape
    d = e // 2
    tab_rows = tab.shape[0]
    info = plsc.get_sparse_core_info()
    nc, ns = info.num_cores, info.num_subcores
    nw = nc * ns
    per_w = nr // nw
    nblk = per_w // NB
    mesh = plsc.VectorSubcoreMesh(core_axis_name="c", subcore_axis_name="s")

    @functools.partial(
        pl.kernel,
        out_type=jax.ShapeDtypeStruct((nr, e), jnp.float32),
        mesh=mesh,
        scratch_types=[
            [pltpu.VMEM((NB, e), jnp.float32)] * DEPTH,
            [pltpu.VMEM((2 * NB, d), jnp.float32)] * DEPTH,
            pltpu.VMEM((2 * per_w,), jnp.float32),
            pltpu.VMEM((2 * per_w,), jnp.int32),
            [pltpu.SemaphoreType.DMA] * DEPTH,
            [pltpu.SemaphoreType.DMA] * DEPTH,
            [pltpu.SemaphoreType.DMA] * DEPTH,
        ],
    )
    def body(x_hbm, posf_hbm, tab_hbm, out_hbm,
             xbufs, pebufs, idxf_all, idx_all, sxs, sgs, sos):
        wid = lax.axis_index("s") * nc + lax.axis_index("c")
        base = wid * per_w
        woff = (wid % TAB_REP) * (tab_rows // TAB_REP)

        pltpu.sync_copy(posf_hbm.at[pl.ds(2 * base, 2 * per_w)], idxf_all)
        for k in range(2 * per_w // 16):
            v = idxf_all[pl.ds(k * 16, 16)]
            idx_all[pl.ds(k * 16, 16)] = (v + 0.5).astype(jnp.int32) + woff

        def add_block(xbuf, pebuf):
            def add_row(r, carry2):
                @plsc.parallel_loop(0, d // 16, unroll=8)
                def add_lo(j):
                    sl = pl.ds(j * 16, 16)
                    xbuf[r, sl] = xbuf[r, sl] + pebuf[2 * r, sl]

                @plsc.parallel_loop(0, d // 16, unroll=8)
                def add_hi(j):
                    sl = pl.ds(d + j * 16, 16)
                    xbuf[r, sl] = xbuf[r, sl] + pebuf[2 * r + 1,
                                                      pl.ds(j * 16, 16)]

                return carry2

            lax.fori_loop(0, NB, add_row, 0)

        def issue_x(i):
            s = i % DEPTH
            return pltpu.async_copy(
                x_hbm.at[pl.ds(base + i * NB, NB)], xbufs[s], sxs[s])

        def issue_g(i):
            s = i % DEPTH
            return pltpu.async_copy(
                tab_hbm.at[idx_all.at[pl.ds(2 * i * NB, 2 * NB)]],
                pebufs[s], sgs[s])

        ldxs = {}
        ldgs = {}
        sts = {}
        for i in range(min(DEPTH, nblk)):
            ldxs[i] = issue_x(i)
            ldgs[i] = issue_g(i)
        for i in range(nblk):
            s = i % DEPTH
            ldxs.pop(i).wait()
            ldgs.pop(i).wait()
            j = i - 1 + DEPTH
            if i >= 1 and j < nblk:
                ldgs[j] = issue_g(j)
            add_block(xbufs[s], pebufs[s])
            sts[i] = pltpu.async_copy(
                xbufs[s], out_hbm.at[pl.ds(base + i * NB, NB)], sos[s])
            if i >= 1 and j < nblk:
                sts.pop(i - 1).wait()
                ldxs[j] = issue_x(j)
        for i in sorted(sts):
            sts.pop(i).wait()

    return body(xr, posf, tab)


def kernel(x, pos, pos_embed):
    b, s1, e = x.shape
    half = e // 2
    nrows = pos_embed.shape[0]
    xr = x.reshape(b * s1, e)
    posf = jnp.pad(
        pos, ((0, 0), (1, 0), (0, 0)), constant_values=float(nrows)
    ).reshape(-1)
    tab = jnp.concatenate(
        [pos_embed, jnp.zeros((1, half), pos_embed.dtype)], axis=0
    )
    tab = jnp.tile(tab, (TAB_REP, 1))
    out = _sc_embed_add(xr, posf, tab)
    return out.reshape(b, s1, e)


# speedup vs baseline: 1.0329x; 1.0329x over previous
import functools

import jax
import jax.numpy as jnp
from jax import lax
from jax.experimental import pallas as pl
from jax.experimental.pallas import tpu as pltpu
from jax.experimental.pallas import tpu_sc as plsc

NB = 8
DEPTH = 3
TAB_REP = 8


def _sc_embed_add(xr, posf, tab):
    nr, e = xr.shape
    d = e // 2
    tab_rows = tab.shape[0]
    info = plsc.get_sparse_core_info()
    nc, ns = info.num_cores, info.num_subcores
    nw = nc * ns
    per_w = nr // nw
    nblk = per_w // NB
    mesh = plsc.VectorSubcoreMesh(core_axis_name="c", subcore_axis_name="s")

    @functools.partial(
        pl.kernel,
        out_type=jax.ShapeDtypeStruct((nr, e), jnp.float32),
        mesh=mesh,
        scratch_types=[
            [pltpu.VMEM((NB, e), jnp.float32)] * DEPTH,
            [pltpu.VMEM((2 * NB, d), jnp.float32)] * DEPTH,
            pltpu.VMEM((2 * per_w,), jnp.float32),
            pltpu.VMEM((2 * per_w,), jnp.int32),
            [pltpu.SemaphoreType.DMA] * DEPTH,
            [pltpu.SemaphoreType.DMA] * DEPTH,
            [pltpu.SemaphoreType.DMA] * DEPTH,
        ],
    )
    def body(x_hbm, posf_hbm, tab_hbm, out_hbm,
             xbufs, pebufs, idxf_all, idx_all, sxs, sgs, sos):
        wid = lax.axis_index("s") * nc + lax.axis_index("c")
        base = wid * per_w
        woff = (wid % TAB_REP) * (tab_rows // TAB_REP)

        pltpu.sync_copy(posf_hbm.at[pl.ds(2 * base, 2 * per_w)], idxf_all)
        for k in range(2 * per_w // 16):
            v = idxf_all[pl.ds(k * 16, 16)]
            idx_all[pl.ds(k * 16, 16)] = (v + 0.5).astype(jnp.int32) + woff

        def add_block(xbuf, pebuf):
            def add_row(r, carry2):
                @plsc.parallel_loop(0, d // 16, unroll=8)
                def add_lo(j):
                    sl = pl.ds(j * 16, 16)
                    xbuf[r, sl] = xbuf[r, sl] + pebuf[2 * r, sl]

                @plsc.parallel_loop(0, d // 16, unroll=8)
                def add_hi(j):
                    sl = pl.ds(d + j * 16, 16)
                    xbuf[r, sl] = xbuf[r, sl] + pebuf[2 * r + 1,
                                                      pl.ds(j * 16, 16)]

                return carry2

            lax.fori_loop(0, NB, add_row, 0)

        def issue_x(i):
            s = i % DEPTH
            return pltpu.async_copy(
                x_hbm.at[pl.ds(base + i * NB, NB)], xbufs[s], sxs[s])

        def issue_g(i):
            s = i % DEPTH
            return pltpu.async_copy(
                tab_hbm.at[idx_all.at[pl.ds(2 * i * NB, 2 * NB)]],
                pebufs[s], sgs[s])

        ldxs = {}
        ldgs = {}
        sts = {}
        for i in range(min(DEPTH, nblk)):
            ldxs[i] = issue_x(i)
            ldgs[i] = issue_g(i)
        for i in range(nblk):
            s = i % DEPTH
            ldxs.pop(i).wait()
            ldgs.pop(i).wait()
            j = i - 1 + DEPTH
            if i >= 1 and j < nblk:
                ldgs[j] = issue_g(j)
            add_block(xbufs[s], pebufs[s])
            sts[i] = pltpu.async_copy(
                xbufs[s], out_hbm.at[pl.ds(base + i * NB, NB)], sos[s])
            if i >= 1 and j < nblk:
                sts.pop(i - 1).wait()
                ldxs[j] = issue_x(j)
        for i in sorted(sts):
            sts.pop(i).wait()

    return body(xr, posf, tab)


def kernel(x, pos, pos_embed):
    b, s1, e = x.shape
    half = e // 2
    nrows = pos_embed.shape[0]
    xr = x.reshape(b * s1, e)
    posf = jnp.pad(
        pos, ((0, 0), (1, 0), (0, 0)), constant_values=float(nrows)
    ).reshape(-1)
    tab = jnp.concatenate(
        [pos_embed, jnp.zeros((1, half), pos_embed.dtype)], axis=0
    )
    tab = jnp.tile(tab, (TAB_REP, 1))
    out = _sc_embed_add(xr, posf, tab)
    return out.reshape(b, s1, e)
